# baseline (speedup 1.0000x reference)
import functools

import jax
import jax.numpy as jnp
from jax import lax
from jax.experimental import pallas as pl
from jax.experimental.pallas import tpu as pltpu
from jax.experimental.pallas import tpu_sc as plsc

D = 128
E = 320000
N = 10000
R = 10000
DEG = 32

_NC = 2
_NS = 16
_NW = _NC * _NS

_CH = 80


def _ln(y, gamma, beta):
    mu = jnp.mean(y, axis=-1, keepdims=True)
    var = jnp.mean((y - mu) ** 2, axis=-1, keepdims=True)
    yn = (y - mu) / jnp.sqrt(var + 1e-5)
    return yn * gamma + beta


def _precompute_body(rect_ref, node_ref, w1t_ref, w2t_ref, at_ref, wggt_ref,
                     ggg_ref, bgg_ref, p_ref, q_ref, na_ref, rect_out_ref):
    rect = rect_ref[...]
    node = node_ref[...]
    p_ref[...] = jnp.dot(rect, w1t_ref[...], preferred_element_type=jnp.float32)
    q_ref[...] = jnp.dot(node, w2t_ref[...], preferred_element_type=jnp.float32)
    na_ref[...] = jnp.dot(node, at_ref[...], preferred_element_type=jnp.float32)
    y = jnp.tanh(jnp.dot(rect, wggt_ref[...], preferred_element_type=jnp.float32))
    rect_out_ref[...] = rect + _ln(y, ggg_ref[...], bgg_ref[...])


def _edge_body(bond_ref, pg_ref, qg_ref, w0t_ref, g_ref, b_ref, db_ref):
    pre = jnp.dot(bond_ref[...], w0t_ref[...], preferred_element_type=jnp.float32)
    pre = pre + pg_ref[...] + qg_ref[...]
    db_ref[...] = _ln(jnp.tanh(pre), g_ref[...], b_ref[...])


def _residual_body(bond_ref, db_ref, bout_ref):
    bout_ref[...] = bond_ref[...] + db_ref[...]


def _node_body(g2_ref, coef_ref, node_ref, na_ref, bt_ref, g_ref, b_ref,
               nout_ref):
    acc = g2_ref[0] * coef_ref[0]
    for k in range(1, DEG):
        acc = acc + g2_ref[k] * coef_ref[k]
    agg = acc * (1.0 / DEG)
    pre = na_ref[...] + jnp.dot(agg, bt_ref[...], preferred_element_type=jnp.float32)
    dn = _ln(jnp.tanh(pre), g_ref[...], b_ref[...])
    nout_ref[...] = node_ref[...] + dn


_BN_A = 1000
_BE = 1000
_BN_C = 400


def _tc_precompute(rect, node, w1t, w2t, at, wggt, ggg, bgg):
    nblk = R // _BN_A
    row_spec = pl.BlockSpec((_BN_A, D), lambda i: (i, 0))
    full_spec = pl.BlockSpec((D, D), lambda i: (0, 0))
    vec_spec = pl.BlockSpec((1, D), lambda i: (0, 0))
    return pl.pallas_call(
        _precompute_body,
        grid=(nblk,),
        in_specs=[row_spec, row_spec, full_spec, full_spec, full_spec,
                  full_spec, vec_spec, vec_spec],
        out_specs=[row_spec, row_spec, row_spec, row_spec],
        out_shape=[jax.ShapeDtypeStruct((R, D), jnp.float32)] * 4,
    )(rect, node, w1t, w2t, at, wggt, ggg, bgg)


def _tc_edge(bond, pg, qg, w0t, g, b):
    nblk = E // _BE
    row_spec = pl.BlockSpec((_BE, D), lambda i: (i, 0))
    full_spec = pl.BlockSpec((D, D), lambda i: (0, 0))
    vec_spec = pl.BlockSpec((1, D), lambda i: (0, 0))
    return pl.pallas_call(
        _edge_body,
        grid=(nblk,),
        in_specs=[row_spec, row_spec, row_spec, full_spec, vec_spec, vec_spec],
        out_specs=row_spec,
        out_shape=jax.ShapeDtypeStruct((E, D), jnp.float32),
    )(bond, pg, qg, w0t, g, b)


def _tc_residual(bond, db):
    nblk = E // _BE
    row_spec = pl.BlockSpec((_BE, D), lambda i: (i, 0))
    return pl.pallas_call(
        _residual_body,
        grid=(nblk,),
        in_specs=[row_spec, row_spec],
        out_specs=row_spec,
        out_shape=jax.ShapeDtypeStruct((E, D), jnp.float32),
    )(bond, db)


def _tc_node(g2, coef3, node, na, bt, g, b):
    nblk = N // _BN_C
    g2_spec = pl.BlockSpec((DEG, _BN_C, D), lambda i: (0, i, 0))
    coef_spec = pl.BlockSpec((DEG, _BN_C, 1), lambda i: (0, i, 0))
    row_spec = pl.BlockSpec((_BN_C, D), lambda i: (i, 0))
    full_spec = pl.BlockSpec((D, D), lambda i: (0, 0))
    vec_spec = pl.BlockSpec((1, D), lambda i: (0, 0))
    return pl.pallas_call(
        _node_body,
        grid=(nblk,),
        in_specs=[g2_spec, coef_spec, row_spec, row_spec, full_spec,
                  vec_spec, vec_spec],
        out_specs=row_spec,
        out_shape=jax.ShapeDtypeStruct((N, D), jnp.float32),
    )(g2, coef3, node, na, bt, g, b)


_NCHUNK = (E // _NW) // _CH


@functools.cache
def _sc_kernels():
    mesh = plsc.VectorSubcoreMesh(core_axis_name="c", subcore_axis_name="s",
                                  num_cores=_NC)

    @functools.partial(
        pl.kernel,
        mesh=mesh,
        out_type=[jax.ShapeDtypeStruct((E, D), jnp.float32)] * 2,
        scratch_types=[
            pltpu.VMEM((_NCHUNK, _CH), jnp.int32),
            pltpu.VMEM((_NCHUNK, _CH), jnp.int32),
            pltpu.VMEM((2, _CH, D), jnp.float32),
            pltpu.VMEM((2, _CH, D), jnp.float32),
            pltpu.SemaphoreType.DMA,
            pltpu.SemaphoreType.DMA,
            pltpu.SemaphoreType.DMA,
            pltpu.SemaphoreType.DMA,
        ],
    )
    def _sc_gather_pq(p_hbm, q_hbm, src3, dst3, pg_hbm, qg_hbm,
                      idx_s, idx_d, bufp, bufq, gsp, gsq, ssp, ssq):
        wid = lax.axis_index("s") * _NC + lax.axis_index("c")
        nper = E // _NW
        pltpu.sync_copy(src3.at[wid], idx_s)
        pltpu.sync_copy(dst3.at[wid], idx_d)

        def issue_gather(i, b):
            pltpu.async_copy(p_hbm.at[idx_s.at[i]], bufp.at[b], gsp)
            pltpu.async_copy(q_hbm.at[idx_d.at[i]], bufq.at[b], gsq)

        def wait_gather(b):
            pltpu.make_async_copy(p_hbm.at[pl.ds(0, _CH)], bufp.at[b], gsp).wait()
            pltpu.make_async_copy(q_hbm.at[pl.ds(0, _CH)], bufq.at[b], gsq).wait()

        def wait_scatter(b):
            pltpu.make_async_copy(bufp.at[b], pg_hbm.at[pl.ds(0, _CH)], ssp).wait()
            pltpu.make_async_copy(bufq.at[b], qg_hbm.at[pl.ds(0, _CH)], ssq).wait()

        issue_gather(0, 0)

        def body(i, carry):
            b = lax.rem(i, 2)
            nb = lax.rem(i + 1, 2)

            @pl.when(i + 1 < _NCHUNK)
            def _():
                @pl.when(i >= 1)
                def _():
                    wait_scatter(nb)
                issue_gather(i + 1, nb)

            wait_gather(b)
            base = wid * nper + i * _CH
            pltpu.async_copy(bufp.at[b], pg_hbm.at[pl.ds(base, _CH)], ssp)
            pltpu.async_copy(bufq.at[b], qg_hbm.at[pl.ds(base, _CH)], ssq)
            return carry

        lax.fori_loop(0, _NCHUNK, body, 0)
        wait_scatter(0)
        wait_scatter(1)

    @functools.partial(
        pl.kernel,
        mesh=mesh,
        out_type=jax.ShapeDtypeStruct((DEG * N, D), jnp.float32),
        scratch_types=[
            pltpu.VMEM((_NCHUNK, _CH), jnp.int32),
            pltpu.VMEM((2, _CH, D), jnp.float32),
            pltpu.SemaphoreType.DMA,
            pltpu.SemaphoreType.DMA,
        ],
    )
    def _sc_gather_deg(db_hbm, eidt3, g2_hbm, idx_v, buf, gs, ss):
        wid = lax.axis_index("s") * _NC + lax.axis_index("c")
        pltpu.sync_copy(eidt3.at[wid], idx_v)

        def body(i, carry):
            b = lax.rem(i, 2)
            nb = lax.rem(i + 1, 2)

            @pl.when(i + 1 < _NCHUNK)
            def _():
                @pl.when(i >= 1)
                def _():
                    pltpu.make_async_copy(buf.at[nb], g2_hbm.at[pl.ds(0, _CH)],
                                          ss).wait()
                pltpu.async_copy(db_hbm.at[idx_v.at[i + 1]], buf.at[nb], gs)

            pltpu.make_async_copy(db_hbm.at[pl.ds(0, _CH)], buf.at[b], gs).wait()
            base = wid * N + i * _CH
            pltpu.async_copy(buf.at[b], g2_hbm.at[pl.ds(base, _CH)], ss)
            return carry

        pltpu.async_copy(db_hbm.at[idx_v.at[0]], buf.at[0], gs)
        lax.fori_loop(0, _NCHUNK, body, 0)
        pltpu.make_async_copy(buf.at[0], g2_hbm.at[pl.ds(0, _CH)], ss).wait()
        pltpu.make_async_copy(buf.at[1], g2_hbm.at[pl.ds(0, _CH)], ss).wait()

    return _sc_gather_pq, _sc_gather_deg


def kernel(grid_mesh_bond_embedding, grid_rect_embedding, mesh_node_embedding,
           G2M_edge_id2pair_tensor, G2M_edge_id_of_node_tensor,
           G2M_edge_coef_node_tensor, W_GM2E, g_GM2E, b_GM2E,
           W_E2M, g_E2M, b_E2M, W_G2G, g_G2G, b_G2G):
    bond = grid_mesh_bond_embedding.reshape(E, D)
    rect = grid_rect_embedding.reshape(R, D)
    node = mesh_node_embedding.reshape(N, D)
    src3 = G2M_edge_id2pair_tensor[:, 0].reshape(_NW, _NCHUNK, _CH)
    dst3 = G2M_edge_id2pair_tensor[:, 1].reshape(_NW, _NCHUNK, _CH)
    eidt3 = G2M_edge_id_of_node_tensor.T.reshape(_NW, _NCHUNK, _CH)
    coef3 = G2M_edge_coef_node_tensor.transpose(1, 0, 2)

    w0t = W_GM2E[:, :D].T
    w1t = W_GM2E[:, D:2 * D].T
    w2t = W_GM2E[:, 2 * D:].T
    at = W_E2M[:, :D].T
    bt = W_E2M[:, D:].T
    wggt = W_G2G.T
    g_gm = g_GM2E.reshape(1, D); b_gm = b_GM2E.reshape(1, D)
    g_em = g_E2M.reshape(1, D); b_em = b_E2M.reshape(1, D)
    g_gg = g_G2G.reshape(1, D); b_gg = b_G2G.reshape(1, D)

    sc_gather_pq, sc_gather_deg = _sc_kernels()
    p, q, na, rect_out = _tc_precompute(rect, node, w1t, w2t, at, wggt,
                                        g_gg, b_gg)
    pg, qg = sc_gather_pq(p, q, src3, dst3)
    db = _tc_edge(bond, pg, qg, w0t, g_gm, b_gm)
    g2 = sc_gather_deg(db, eidt3).reshape(DEG, N, D)
    bond_out = _tc_residual(bond, db)
    node_out = _tc_node(g2, coef3, node, na, bt, g_em, b_em)

    return (bond_out.reshape(1, E, D),
            rect_out.reshape(1, R, D),
            node_out.reshape(1, N, D))


# speedup vs baseline: 1.3688x; 1.3688x over previous
import functools

import jax
import jax.numpy as jnp
from jax import lax
from jax.experimental import pallas as pl
from jax.experimental.pallas import tpu as pltpu
from jax.experimental.pallas import tpu_sc as plsc

D = 128
E = 320000
N = 10000
R = 10000
DEG = 32

_NC = 2
_NS = 16
_NW = _NC * _NS

_CH = 80


def _ln(y, gamma, beta):
    mu = jnp.mean(y, axis=-1, keepdims=True)
    var = jnp.mean((y - mu) ** 2, axis=-1, keepdims=True)
    yn = (y - mu) / jnp.sqrt(var + 1e-5)
    return yn * gamma + beta


def _precompute_body(rect_ref, node_ref, w1t_ref, w2t_ref, at_ref, wggt_ref,
                     ggg_ref, bgg_ref, p_ref, q_ref, na_ref, rect_out_ref):
    rect = rect_ref[...]
    node = node_ref[...]
    p_ref[...] = jnp.dot(rect, w1t_ref[...], preferred_element_type=jnp.float32)
    q_ref[...] = jnp.dot(node, w2t_ref[...], preferred_element_type=jnp.float32)
    na_ref[...] = jnp.dot(node, at_ref[...], preferred_element_type=jnp.float32)
    y = jnp.tanh(jnp.dot(rect, wggt_ref[...], preferred_element_type=jnp.float32))
    rect_out_ref[...] = rect + _ln(y, ggg_ref[...], bgg_ref[...])


def _edge_body(bond_ref, pg_ref, qg_ref, w0t_ref, g_ref, b_ref,
               db_ref, bout_ref):
    pre = jnp.dot(bond_ref[...], w0t_ref[...], preferred_element_type=jnp.float32)
    pre = pre + pg_ref[...] + qg_ref[...]
    db = _ln(jnp.tanh(pre), g_ref[...], b_ref[...])
    db_ref[...] = db
    bout_ref[...] = bond_ref[...] + db


def _node_body(g2_ref, coef_ref, node_ref, na_ref, bt_ref, g_ref, b_ref,
               nout_ref):
    acc = g2_ref[0] * coef_ref[0]
    for k in range(1, DEG):
        acc = acc + g2_ref[k] * coef_ref[k]
    agg = acc * (1.0 / DEG)
    pre = na_ref[...] + jnp.dot(agg, bt_ref[...], preferred_element_type=jnp.float32)
    dn = _ln(jnp.tanh(pre), g_ref[...], b_ref[...])
    nout_ref[...] = node_ref[...] + dn


_BN_A = 1000
_BE = 2000
_BN_C = 400


def _tc_precompute(rect, node, w1t, w2t, at, wggt, ggg, bgg):
    nblk = R // _BN_A
    row_spec = pl.BlockSpec((_BN_A, D), lambda i: (i, 0))
    full_spec = pl.BlockSpec((D, D), lambda i: (0, 0))
    vec_spec = pl.BlockSpec((1, D), lambda i: (0, 0))
    return pl.pallas_call(
        _precompute_body,
        grid=(nblk,),
        in_specs=[row_spec, row_spec, full_spec, full_spec, full_spec,
                  full_spec, vec_spec, vec_spec],
        out_specs=[row_spec, row_spec, row_spec, row_spec],
        out_shape=[jax.ShapeDtypeStruct((R, D), jnp.float32)] * 4,
    )(rect, node, w1t, w2t, at, wggt, ggg, bgg)


def _tc_edge(bond, pg, qg, w0t, g, b):
    nblk = E // _BE
    row_spec = pl.BlockSpec((_BE, D), lambda i: (i, 0))
    full_spec = pl.BlockSpec((D, D), lambda i: (0, 0))
    vec_spec = pl.BlockSpec((1, D), lambda i: (0, 0))
    return pl.pallas_call(
        _edge_body,
        grid=(nblk,),
        in_specs=[row_spec, row_spec, row_spec, full_spec, vec_spec, vec_spec],
        out_specs=[row_spec, row_spec],
        out_shape=[jax.ShapeDtypeStruct((E, D), jnp.float32)] * 2,
    )(bond, pg, qg, w0t, g, b)


def _tc_node(g2, coef3, node, na, bt, g, b):
    nblk = N // _BN_C
    g2_spec = pl.BlockSpec((DEG, _BN_C, D), lambda i: (0, i, 0))
    coef_spec = pl.BlockSpec((DEG, _BN_C, 1), lambda i: (0, i, 0))
    row_spec = pl.BlockSpec((_BN_C, D), lambda i: (i, 0))
    full_spec = pl.BlockSpec((D, D), lambda i: (0, 0))
    vec_spec = pl.BlockSpec((1, D), lambda i: (0, 0))
    return pl.pallas_call(
        _node_body,
        grid=(nblk,),
        in_specs=[g2_spec, coef_spec, row_spec, row_spec, full_spec,
                  vec_spec, vec_spec],
        out_specs=row_spec,
        out_shape=jax.ShapeDtypeStruct((N, D), jnp.float32),
    )(g2, coef3, node, na, bt, g, b)


_NCHUNK = (E // _NW) // _CH


@functools.cache
def _sc_kernels():
    mesh = plsc.VectorSubcoreMesh(core_axis_name="c", subcore_axis_name="s",
                                  num_cores=_NC)

    @functools.partial(
        pl.kernel,
        mesh=mesh,
        out_type=[jax.ShapeDtypeStruct((E, D), jnp.float32)] * 2,
        scratch_types=[
            pltpu.VMEM((_NCHUNK, _CH), jnp.int32),
            pltpu.VMEM((_NCHUNK, _CH), jnp.int32),
            pltpu.VMEM((2, _CH, D), jnp.float32),
            pltpu.VMEM((2, _CH, D), jnp.float32),
            pltpu.SemaphoreType.DMA,
            pltpu.SemaphoreType.DMA,
            pltpu.SemaphoreType.DMA,
            pltpu.SemaphoreType.DMA,
        ],
    )
    def _sc_gather_pq(p_hbm, q_hbm, src3, dst3, pg_hbm, qg_hbm,
                      idx_s, idx_d, bufp, bufq, gsp, gsq, ssp, ssq):
        wid = lax.axis_index("s") * _NC + lax.axis_index("c")
        nper = E // _NW
        pltpu.sync_copy(src3.at[wid], idx_s)
        pltpu.sync_copy(dst3.at[wid], idx_d)

        def issue_gather(i, b):
            pltpu.async_copy(p_hbm.at[idx_s.at[i]], bufp.at[b], gsp)
            pltpu.async_copy(q_hbm.at[idx_d.at[i]], bufq.at[b], gsq)

        def wait_gather(b):
            pltpu.make_async_copy(p_hbm.at[pl.ds(0, _CH)], bufp.at[b], gsp).wait()
            pltpu.make_async_copy(q_hbm.at[pl.ds(0, _CH)], bufq.at[b], gsq).wait()

        def wait_scatter(b):
            pltpu.make_async_copy(bufp.at[b], pg_hbm.at[pl.ds(0, _CH)], ssp).wait()
            pltpu.make_async_copy(bufq.at[b], qg_hbm.at[pl.ds(0, _CH)], ssq).wait()

        issue_gather(0, 0)

        def body(i, carry):
            b = lax.rem(i, 2)
            nb = lax.rem(i + 1, 2)

            @pl.when(i + 1 < _NCHUNK)
            def _():
                @pl.when(i >= 1)
                def _():
                    wait_scatter(nb)
                issue_gather(i + 1, nb)

            wait_gather(b)
            base = wid * nper + i * _CH
            pltpu.async_copy(bufp.at[b], pg_hbm.at[pl.ds(base, _CH)], ssp)
            pltpu.async_copy(bufq.at[b], qg_hbm.at[pl.ds(base, _CH)], ssq)
            return carry

        lax.fori_loop(0, _NCHUNK, body, 0)
        wait_scatter(0)
        wait_scatter(1)

    @functools.partial(
        pl.kernel,
        mesh=mesh,
        out_type=jax.ShapeDtypeStruct((DEG * N, D), jnp.float32),
        scratch_types=[
            pltpu.VMEM((_NCHUNK, _CH), jnp.int32),
            pltpu.VMEM((2, _CH, D), jnp.float32),
            pltpu.SemaphoreType.DMA,
            pltpu.SemaphoreType.DMA,
        ],
    )
    def _sc_gather_deg(db_hbm, eidt3, g2_hbm, idx_v, buf, gs, ss):
        wid = lax.axis_index("s") * _NC + lax.axis_index("c")
        pltpu.sync_copy(eidt3.at[wid], idx_v)

        def body(i, carry):
            b = lax.rem(i, 2)
            nb = lax.rem(i + 1, 2)

            @pl.when(i + 1 < _NCHUNK)
            def _():
                @pl.when(i >= 1)
                def _():
                    pltpu.make_async_copy(buf.at[nb], g2_hbm.at[pl.ds(0, _CH)],
                                          ss).wait()
                pltpu.async_copy(db_hbm.at[idx_v.at[i + 1]], buf.at[nb], gs)

            pltpu.make_async_copy(db_hbm.at[pl.ds(0, _CH)], buf.at[b], gs).wait()
            base = wid * N + i * _CH
            pltpu.async_copy(buf.at[b], g2_hbm.at[pl.ds(base, _CH)], ss)
            return carry

        pltpu.async_copy(db_hbm.at[idx_v.at[0]], buf.at[0], gs)
        lax.fori_loop(0, _NCHUNK, body, 0)
        pltpu.make_async_copy(buf.at[0], g2_hbm.at[pl.ds(0, _CH)], ss).wait()
        pltpu.make_async_copy(buf.at[1], g2_hbm.at[pl.ds(0, _CH)], ss).wait()

    return _sc_gather_pq, _sc_gather_deg


def kernel(grid_mesh_bond_embedding, grid_rect_embedding, mesh_node_embedding,
           G2M_edge_id2pair_tensor, G2M_edge_id_of_node_tensor,
           G2M_edge_coef_node_tensor, W_GM2E, g_GM2E, b_GM2E,
           W_E2M, g_E2M, b_E2M, W_G2G, g_G2G, b_G2G):
    bond = grid_mesh_bond_embedding.reshape(E, D)
    rect = grid_rect_embedding.reshape(R, D)
    node = mesh_node_embedding.reshape(N, D)
    src3 = G2M_edge_id2pair_tensor[:, 0].reshape(_NW, _NCHUNK, _CH)
    dst3 = G2M_edge_id2pair_tensor[:, 1].reshape(_NW, _NCHUNK, _CH)
    eidt3 = G2M_edge_id_of_node_tensor.T.reshape(_NW, _NCHUNK, _CH)
    coef3 = G2M_edge_coef_node_tensor.transpose(1, 0, 2)

    w0t = W_GM2E[:, :D].T
    w1t = W_GM2E[:, D:2 * D].T
    w2t = W_GM2E[:, 2 * D:].T
    at = W_E2M[:, :D].T
    bt = W_E2M[:, D:].T
    wggt = W_G2G.T
    g_gm = g_GM2E.reshape(1, D); b_gm = b_GM2E.reshape(1, D)
    g_em = g_E2M.reshape(1, D); b_em = b_E2M.reshape(1, D)
    g_gg = g_G2G.reshape(1, D); b_gg = b_G2G.reshape(1, D)

    sc_gather_pq, sc_gather_deg = _sc_kernels()
    p, q, na, rect_out = _tc_precompute(rect, node, w1t, w2t, at, wggt,
                                        g_gg, b_gg)
    pg, qg = sc_gather_pq(p, q, src3, dst3)
    db, bond_out = _tc_edge(bond, pg, qg, w0t, g_gm, b_gm)
    g2 = sc_gather_deg(db, eidt3).reshape(DEG, N, D)
    node_out = _tc_node(g2, coef3, node, na, bt, g_em, b_em)

    return (bond_out.reshape(1, E, D),
            rect_out.reshape(1, R, D),
            node_out.reshape(1, N, D))


# speedup vs baseline: 1.4825x; 1.0831x over previous
import functools

import jax
import jax.numpy as jnp
from jax import lax
from jax.experimental import pallas as pl
from jax.experimental.pallas import tpu as pltpu
from jax.experimental.pallas import tpu_sc as plsc

D = 128
E = 320000
N = 10000
R = 10000
DEG = 32

_NC = 2
_NS = 16
_NW = _NC * _NS

_CH = 80


def _ln(y, gamma, beta):
    mu = jnp.mean(y, axis=-1, keepdims=True)
    var = jnp.mean((y - mu) ** 2, axis=-1, keepdims=True)
    yn = (y - mu) / jnp.sqrt(var + 1e-5)
    return yn * gamma + beta


def _precompute_body(rect_ref, node_ref, w1t_ref, w2t_ref, at_ref, wggt_ref,
                     ggg_ref, bgg_ref, p_ref, q_ref, na_ref, rect_out_ref):
    rect = rect_ref[...]
    node = node_ref[...]
    p_ref[...] = jnp.dot(rect, w1t_ref[...], preferred_element_type=jnp.float32)
    q_ref[...] = jnp.dot(node, w2t_ref[...], preferred_element_type=jnp.float32)
    na_ref[...] = jnp.dot(node, at_ref[...], preferred_element_type=jnp.float32)
    y = jnp.tanh(jnp.dot(rect, wggt_ref[...], preferred_element_type=jnp.float32))
    rect_out_ref[...] = rect + _ln(y, ggg_ref[...], bgg_ref[...])


def _edge_body(bond_ref, s_ref, w0t_ref, g_ref, b_ref,
               db_ref, bout_ref):
    pre = jnp.dot(bond_ref[...], w0t_ref[...], preferred_element_type=jnp.float32)
    pre = pre + s_ref[...]
    db = _ln(jnp.tanh(pre), g_ref[...], b_ref[...])
    db_ref[...] = db
    bout_ref[...] = bond_ref[...] + db


def _node_body(g2_ref, coef_ref, node_ref, na_ref, bt_ref, g_ref, b_ref,
               nout_ref):
    acc = g2_ref[0] * coef_ref[0]
    for k in range(1, DEG):
        acc = acc + g2_ref[k] * coef_ref[k]
    agg = acc * (1.0 / DEG)
    pre = na_ref[...] + jnp.dot(agg, bt_ref[...], preferred_element_type=jnp.float32)
    dn = _ln(jnp.tanh(pre), g_ref[...], b_ref[...])
    nout_ref[...] = node_ref[...] + dn


_BN_A = 1000
_BE = 2000
_BN_C = 400


def _tc_precompute(rect, node, w1t, w2t, at, wggt, ggg, bgg):
    nblk = R // _BN_A
    row_spec = pl.BlockSpec((_BN_A, D), lambda i: (i, 0))
    full_spec = pl.BlockSpec((D, D), lambda i: (0, 0))
    vec_spec = pl.BlockSpec((1, D), lambda i: (0, 0))
    return pl.pallas_call(
        _precompute_body,
        grid=(nblk,),
        in_specs=[row_spec, row_spec, full_spec, full_spec, full_spec,
                  full_spec, vec_spec, vec_spec],
        out_specs=[row_spec, row_spec, row_spec, row_spec],
        out_shape=[jax.ShapeDtypeStruct((R, D), jnp.float32)] * 4,
    )(rect, node, w1t, w2t, at, wggt, ggg, bgg)


def _tc_edge(bond, s, w0t, g, b):
    nblk = E // _BE
    row_spec = pl.BlockSpec((_BE, D), lambda i: (i, 0))
    full_spec = pl.BlockSpec((D, D), lambda i: (0, 0))
    vec_spec = pl.BlockSpec((1, D), lambda i: (0, 0))
    return pl.pallas_call(
        _edge_body,
        grid=(nblk,),
        in_specs=[row_spec, row_spec, full_spec, vec_spec, vec_spec],
        out_specs=[row_spec, row_spec],
        out_shape=[jax.ShapeDtypeStruct((E, D), jnp.float32)] * 2,
    )(bond, s, w0t, g, b)


def _tc_node(g2, coef3, node, na, bt, g, b):
    nblk = N // _BN_C
    g2_spec = pl.BlockSpec((DEG, _BN_C, D), lambda i: (0, i, 0))
    coef_spec = pl.BlockSpec((DEG, _BN_C, 1), lambda i: (0, i, 0))
    row_spec = pl.BlockSpec((_BN_C, D), lambda i: (i, 0))
    full_spec = pl.BlockSpec((D, D), lambda i: (0, 0))
    vec_spec = pl.BlockSpec((1, D), lambda i: (0, 0))
    return pl.pallas_call(
        _node_body,
        grid=(nblk,),
        in_specs=[g2_spec, coef_spec, row_spec, row_spec, full_spec,
                  vec_spec, vec_spec],
        out_specs=row_spec,
        out_shape=jax.ShapeDtypeStruct((N, D), jnp.float32),
    )(g2, coef3, node, na, bt, g, b)


_NCHUNK = (E // _NW) // _CH


@functools.cache
def _sc_kernels():
    mesh = plsc.VectorSubcoreMesh(core_axis_name="c", subcore_axis_name="s",
                                  num_cores=_NC)

    @functools.partial(
        pl.kernel,
        mesh=mesh,
        out_type=jax.ShapeDtypeStruct((E, D), jnp.float32),
        scratch_types=[
            pltpu.VMEM((_NCHUNK, _CH), jnp.int32),
            pltpu.VMEM((_NCHUNK, _CH), jnp.int32),
            pltpu.VMEM((_CH, D), jnp.float32),
            pltpu.VMEM((_CH, D), jnp.float32),
            pltpu.VMEM((_CH, D), jnp.float32),
            pltpu.VMEM((_CH, D), jnp.float32),
            pltpu.SemaphoreType.DMA,
            pltpu.SemaphoreType.DMA,
            pltpu.SemaphoreType.DMA,
        ],
    )
    def _sc_gather_pq(p_hbm, q_hbm, src3, dst3, s_hbm,
                      idx_s, idx_d, bp0, bq0, bp1, bq1, gsp, gsq, ss):
        wid = lax.axis_index("s") * _NC + lax.axis_index("c")
        nper = E // _NW
        pltpu.sync_copy(src3.at[wid], idx_s)
        pltpu.sync_copy(dst3.at[wid], idx_d)

        def issue_gather(i, bp, bq):
            pltpu.async_copy(p_hbm.at[idx_s.at[i]], bp, gsp)
            pltpu.async_copy(q_hbm.at[idx_d.at[i]], bq, gsq)

        def wait_gather(bp, bq):
            pltpu.make_async_copy(p_hbm.at[pl.ds(0, _CH)], bp, gsp).wait()
            pltpu.make_async_copy(q_hbm.at[pl.ds(0, _CH)], bq, gsq).wait()

        def wait_scatter():
            pltpu.make_async_copy(bq0, s_hbm.at[pl.ds(0, _CH)], ss).wait()

        def step(i, bp, bq, bpn, bqn):
            @pl.when(i + 1 < _NCHUNK)
            def _():
                @pl.when(i >= 1)
                def _():
                    wait_scatter()
                issue_gather(i + 1, bpn, bqn)

            wait_gather(bp, bq)

            @plsc.parallel_loop(0, _CH, 1, unroll=2)
            def _(r):
                for j in range(D // 16):
                    sl = pl.ds(j * 16, 16)
                    bq[r, sl] = bq[r, sl] + bp[r, sl]

            pltpu.async_copy(bq, s_hbm.at[pl.ds(wid * nper + i * _CH, _CH)],
                             ss)

        issue_gather(0, bp0, bq0)

        def body(i2, carry):
            step(2 * i2, bp0, bq0, bp1, bq1)
            step(2 * i2 + 1, bp1, bq1, bp0, bq0)
            return carry

        lax.fori_loop(0, _NCHUNK // 2, body, 0)
        step(_NCHUNK - 1, bp0, bq0, bp1, bq1)
        wait_scatter()
        wait_scatter()

    @functools.partial(
        pl.kernel,
        mesh=mesh,
        out_type=jax.ShapeDtypeStruct((DEG * N, D), jnp.float32),
        scratch_types=[
            pltpu.VMEM((_NCHUNK, _CH), jnp.int32),
            pltpu.VMEM((2, _CH, D), jnp.float32),
            pltpu.SemaphoreType.DMA,
            pltpu.SemaphoreType.DMA,
        ],
    )
    def _sc_gather_deg(db_hbm, eidt3, g2_hbm, idx_v, buf, gs, ss):
        wid = lax.axis_index("s") * _NC + lax.axis_index("c")
        pltpu.sync_copy(eidt3.at[wid], idx_v)

        def body(i, carry):
            b = lax.rem(i, 2)
            nb = lax.rem(i + 1, 2)

            @pl.when(i + 1 < _NCHUNK)
            def _():
                @pl.when(i >= 1)
                def _():
                    pltpu.make_async_copy(buf.at[nb], g2_hbm.at[pl.ds(0, _CH)],
                                          ss).wait()
                pltpu.async_copy(db_hbm.at[idx_v.at[i + 1]], buf.at[nb], gs)

            pltpu.make_async_copy(db_hbm.at[pl.ds(0, _CH)], buf.at[b], gs).wait()
            base = wid * N + i * _CH
            pltpu.async_copy(buf.at[b], g2_hbm.at[pl.ds(base, _CH)], ss)
            return carry

        pltpu.async_copy(db_hbm.at[idx_v.at[0]], buf.at[0], gs)
        lax.fori_loop(0, _NCHUNK, body, 0)
        pltpu.make_async_copy(buf.at[0], g2_hbm.at[pl.ds(0, _CH)], ss).wait()
        pltpu.make_async_copy(buf.at[1], g2_hbm.at[pl.ds(0, _CH)], ss).wait()

    return _sc_gather_pq, _sc_gather_deg


def kernel(grid_mesh_bond_embedding, grid_rect_embedding, mesh_node_embedding,
           G2M_edge_id2pair_tensor, G2M_edge_id_of_node_tensor,
           G2M_edge_coef_node_tensor, W_GM2E, g_GM2E, b_GM2E,
           W_E2M, g_E2M, b_E2M, W_G2G, g_G2G, b_G2G):
    bond = grid_mesh_bond_embedding.reshape(E, D)
    rect = grid_rect_embedding.reshape(R, D)
    node = mesh_node_embedding.reshape(N, D)
    src3 = G2M_edge_id2pair_tensor[:, 0].reshape(_NW, _NCHUNK, _CH)
    dst3 = G2M_edge_id2pair_tensor[:, 1].reshape(_NW, _NCHUNK, _CH)
    eidt3 = G2M_edge_id_of_node_tensor.T.reshape(_NW, _NCHUNK, _CH)
    coef3 = G2M_edge_coef_node_tensor.transpose(1, 0, 2)

    w0t = W_GM2E[:, :D].T
    w1t = W_GM2E[:, D:2 * D].T
    w2t = W_GM2E[:, 2 * D:].T
    at = W_E2M[:, :D].T
    bt = W_E2M[:, D:].T
    wggt = W_G2G.T
    g_gm = g_GM2E.reshape(1, D); b_gm = b_GM2E.reshape(1, D)
    g_em = g_E2M.reshape(1, D); b_em = b_E2M.reshape(1, D)
    g_gg = g_G2G.reshape(1, D); b_gg = b_G2G.reshape(1, D)

    sc_gather_pq, sc_gather_deg = _sc_kernels()
    p, q, na, rect_out = _tc_precompute(rect, node, w1t, w2t, at, wggt,
                                        g_gg, b_gg)
    s = sc_gather_pq(p, q, src3, dst3)
    db, bond_out = _tc_edge(bond, s, w0t, g_gm, b_gm)
    g2 = sc_gather_deg(db, eidt3).reshape(DEG, N, D)
    node_out = _tc_node(g2, coef3, node, na, bt, g_em, b_em)

    return (bond_out.reshape(1, E, D),
            rect_out.reshape(1, R, D),
            node_out.reshape(1, N, D))


# speedup vs baseline: 1.5757x; 1.0629x over previous
import functools

import jax
import jax.numpy as jnp
from jax import lax
from jax.experimental import pallas as pl
from jax.experimental.pallas import tpu as pltpu
from jax.experimental.pallas import tpu_sc as plsc

D = 128
E = 320000
N = 10000
R = 10000
DEG = 32

_NC = 2
_NS = 16
_NW = _NC * _NS

_CH = 80


def _ln(y, gamma, beta):
    mu = jnp.mean(y, axis=-1, keepdims=True)
    var = jnp.mean((y - mu) ** 2, axis=-1, keepdims=True)
    yn = (y - mu) / jnp.sqrt(var + 1e-5)
    return yn * gamma + beta


def _precompute_body(rect_ref, node_ref, w1t_ref, w2t_ref, at_ref, wggt_ref,
                     ggg_ref, bgg_ref, p_ref, q_ref, na_ref, rect_out_ref):
    rect = rect_ref[...]
    node = node_ref[...]
    p_ref[...] = jnp.dot(rect, w1t_ref[...], preferred_element_type=jnp.float32)
    q_ref[...] = jnp.dot(node, w2t_ref[...], preferred_element_type=jnp.float32)
    na_ref[...] = jnp.dot(node, at_ref[...], preferred_element_type=jnp.float32)
    y = jnp.tanh(jnp.dot(rect, wggt_ref[...], preferred_element_type=jnp.float32))
    rect_out_ref[...] = rect + _ln(y, ggg_ref[...], bgg_ref[...])


def _edge_body(bond_ref, s_ref, w0t_ref, g_ref, b_ref,
               db_ref, bout_ref):
    pre = jnp.dot(bond_ref[...], w0t_ref[...], preferred_element_type=jnp.float32)
    pre = pre + s_ref[...]
    db = _ln(jnp.tanh(pre), g_ref[...], b_ref[...])
    db_ref[...] = db
    bout_ref[...] = bond_ref[...] + db


def _node_body(g2_ref, coef_ref, node_ref, na_ref, bt_ref, g_ref, b_ref,
               nout_ref):
    acc = g2_ref[0] * coef_ref[0]
    for k in range(1, DEG):
        acc = acc + g2_ref[k] * coef_ref[k]
    agg = acc * (1.0 / DEG)
    pre = na_ref[...] + jnp.dot(agg, bt_ref[...], preferred_element_type=jnp.float32)
    dn = _ln(jnp.tanh(pre), g_ref[...], b_ref[...])
    nout_ref[...] = node_ref[...] + dn


_BN_A = 1000
_BE = 4000
_BN_C = 400


def _tc_precompute(rect, node, w1t, w2t, at, wggt, ggg, bgg):
    nblk = R // _BN_A
    row_spec = pl.BlockSpec((_BN_A, D), lambda i: (i, 0))
    full_spec = pl.BlockSpec((D, D), lambda i: (0, 0))
    vec_spec = pl.BlockSpec((1, D), lambda i: (0, 0))
    return pl.pallas_call(
        _precompute_body,
        grid=(nblk,),
        in_specs=[row_spec, row_spec, full_spec, full_spec, full_spec,
                  full_spec, vec_spec, vec_spec],
        out_specs=[row_spec, row_spec, row_spec, row_spec],
        out_shape=[jax.ShapeDtypeStruct((R, D), jnp.float32)] * 4,
    )(rect, node, w1t, w2t, at, wggt, ggg, bgg)


def _tc_edge(bond, s, w0t, g, b):
    nblk = E // _BE
    row_spec = pl.BlockSpec((_BE, D), lambda i: (i, 0))
    full_spec = pl.BlockSpec((D, D), lambda i: (0, 0))
    vec_spec = pl.BlockSpec((1, D), lambda i: (0, 0))
    return pl.pallas_call(
        _edge_body,
        grid=(nblk,),
        in_specs=[row_spec, row_spec, full_spec, vec_spec, vec_spec],
        out_specs=[row_spec, row_spec],
        out_shape=[jax.ShapeDtypeStruct((E, D), jnp.float32)] * 2,
    )(bond, s, w0t, g, b)


def _tc_node(g2, coef3, node, na, bt, g, b):
    nblk = N // _BN_C
    g2_spec = pl.BlockSpec((DEG, _BN_C, D), lambda i: (0, i, 0))
    coef_spec = pl.BlockSpec((DEG, _BN_C, 1), lambda i: (0, i, 0))
    row_spec = pl.BlockSpec((_BN_C, D), lambda i: (i, 0))
    full_spec = pl.BlockSpec((D, D), lambda i: (0, 0))
    vec_spec = pl.BlockSpec((1, D), lambda i: (0, 0))
    return pl.pallas_call(
        _node_body,
        grid=(nblk,),
        in_specs=[g2_spec, coef_spec, row_spec, row_spec, full_spec,
                  vec_spec, vec_spec],
        out_specs=row_spec,
        out_shape=jax.ShapeDtypeStruct((N, D), jnp.float32),
    )(g2, coef3, node, na, bt, g, b)


_NCHUNK = (E // _NW) // _CH


@functools.cache
def _sc_kernels():
    mesh = plsc.VectorSubcoreMesh(core_axis_name="c", subcore_axis_name="s",
                                  num_cores=_NC)

    @functools.partial(
        pl.kernel,
        mesh=mesh,
        out_type=jax.ShapeDtypeStruct((E, D), jnp.float32),
        scratch_types=[
            pltpu.VMEM((_NCHUNK, _CH), jnp.int32),
            pltpu.VMEM((_NCHUNK, _CH), jnp.int32),
            pltpu.VMEM((_CH, D), jnp.float32),
            pltpu.VMEM((_CH, D), jnp.float32),
            pltpu.VMEM((_CH, D), jnp.float32),
            pltpu.VMEM((_CH, D), jnp.float32),
            pltpu.SemaphoreType.DMA,
            pltpu.SemaphoreType.DMA,
            pltpu.SemaphoreType.DMA,
        ],
    )
    def _sc_gather_pq(p_hbm, q_hbm, src3, dst3, s_hbm,
                      idx_s, idx_d, bp0, bq0, bp1, bq1, gsp, gsq, ss):
        wid = lax.axis_index("s") * _NC + lax.axis_index("c")
        nper = E // _NW
        pltpu.sync_copy(src3.at[wid], idx_s)
        pltpu.sync_copy(dst3.at[wid], idx_d)

        def issue_gather(i, bp, bq):
            pltpu.async_copy(p_hbm.at[idx_s.at[i]], bp, gsp)
            pltpu.async_copy(q_hbm.at[idx_d.at[i]], bq, gsq)

        def wait_gather(bp, bq):
            pltpu.make_async_copy(p_hbm.at[pl.ds(0, _CH)], bp, gsp).wait()
            pltpu.make_async_copy(q_hbm.at[pl.ds(0, _CH)], bq, gsq).wait()

        def wait_scatter():
            pltpu.make_async_copy(bq0, s_hbm.at[pl.ds(0, _CH)], ss).wait()

        def step(i, bp, bq, bpn, bqn):
            @pl.when(i + 1 < _NCHUNK)
            def _():
                @pl.when(i >= 1)
                def _():
                    wait_scatter()
                issue_gather(i + 1, bpn, bqn)

            wait_gather(bp, bq)

            @plsc.parallel_loop(0, _CH, 1, unroll=2)
            def _(r):
                for j in range(D // 16):
                    sl = pl.ds(j * 16, 16)
                    bq[r, sl] = bq[r, sl] + bp[r, sl]

            pltpu.async_copy(bq, s_hbm.at[pl.ds(wid * nper + i * _CH, _CH)],
                             ss)

        issue_gather(0, bp0, bq0)

        def body(i2, carry):
            step(2 * i2, bp0, bq0, bp1, bq1)
            step(2 * i2 + 1, bp1, bq1, bp0, bq0)
            return carry

        lax.fori_loop(0, _NCHUNK // 2, body, 0)
        step(_NCHUNK - 1, bp0, bq0, bp1, bq1)
        wait_scatter()
        wait_scatter()

    @functools.partial(
        pl.kernel,
        mesh=mesh,
        out_type=jax.ShapeDtypeStruct((DEG * N, D), jnp.float32),
        scratch_types=[
            pltpu.VMEM((_NCHUNK, _CH), jnp.int32),
            pltpu.VMEM((2, _CH, D), jnp.float32),
            pltpu.SemaphoreType.DMA,
            pltpu.SemaphoreType.DMA,
        ],
    )
    def _sc_gather_deg(db_hbm, eidt3, g2_hbm, idx_v, buf, gs, ss):
        wid = lax.axis_index("s") * _NC + lax.axis_index("c")
        pltpu.sync_copy(eidt3.at[wid], idx_v)

        def body(i, carry):
            b = lax.rem(i, 2)
            nb = lax.rem(i + 1, 2)

            @pl.when(i + 1 < _NCHUNK)
            def _():
                @pl.when(i >= 1)
                def _():
                    pltpu.make_async_copy(buf.at[nb], g2_hbm.at[pl.ds(0, _CH)],
                                          ss).wait()
                pltpu.async_copy(db_hbm.at[idx_v.at[i + 1]], buf.at[nb], gs)

            pltpu.make_async_copy(db_hbm.at[pl.ds(0, _CH)], buf.at[b], gs).wait()
            base = wid * N + i * _CH
            pltpu.async_copy(buf.at[b], g2_hbm.at[pl.ds(base, _CH)], ss)
            return carry

        pltpu.async_copy(db_hbm.at[idx_v.at[0]], buf.at[0], gs)
        lax.fori_loop(0, _NCHUNK, body, 0)
        pltpu.make_async_copy(buf.at[0], g2_hbm.at[pl.ds(0, _CH)], ss).wait()
        pltpu.make_async_copy(buf.at[1], g2_hbm.at[pl.ds(0, _CH)], ss).wait()

    return _sc_gather_pq, _sc_gather_deg


def kernel(grid_mesh_bond_embedding, grid_rect_embedding, mesh_node_embedding,
           G2M_edge_id2pair_tensor, G2M_edge_id_of_node_tensor,
           G2M_edge_coef_node_tensor, W_GM2E, g_GM2E, b_GM2E,
           W_E2M, g_E2M, b_E2M, W_G2G, g_G2G, b_G2G):
    bond = grid_mesh_bond_embedding.reshape(E, D)
    rect = grid_rect_embedding.reshape(R, D)
    node = mesh_node_embedding.reshape(N, D)
    src3 = G2M_edge_id2pair_tensor[:, 0].reshape(_NW, _NCHUNK, _CH)
    dst3 = G2M_edge_id2pair_tensor[:, 1].reshape(_NW, _NCHUNK, _CH)
    eidt3 = G2M_edge_id_of_node_tensor.T.reshape(_NW, _NCHUNK, _CH)
    coef3 = G2M_edge_coef_node_tensor.transpose(1, 0, 2)

    w0t = W_GM2E[:, :D].T
    w1t = W_GM2E[:, D:2 * D].T
    w2t = W_GM2E[:, 2 * D:].T
    at = W_E2M[:, :D].T
    bt = W_E2M[:, D:].T
    wggt = W_G2G.T
    g_gm = g_GM2E.reshape(1, D); b_gm = b_GM2E.reshape(1, D)
    g_em = g_E2M.reshape(1, D); b_em = b_E2M.reshape(1, D)
    g_gg = g_G2G.reshape(1, D); b_gg = b_G2G.reshape(1, D)

    sc_gather_pq, sc_gather_deg = _sc_kernels()
    p, q, na, rect_out = _tc_precompute(rect, node, w1t, w2t, at, wggt,
                                        g_gg, b_gg)
    s = sc_gather_pq(p, q, src3, dst3)
    db, bond_out = _tc_edge(bond, s, w0t, g_gm, b_gm)
    g2 = sc_gather_deg(db, eidt3).reshape(DEG, N, D)
    node_out = _tc_node(g2, coef3, node, na, bt, g_em, b_em)

    return (bond_out.reshape(1, E, D),
            rect_out.reshape(1, R, D),
            node_out.reshape(1, N, D))


# speedup vs baseline: 1.6257x; 1.0317x over previous
import functools

import jax
import jax.numpy as jnp
from jax import lax
from jax.experimental import pallas as pl
from jax.experimental.pallas import tpu as pltpu
from jax.experimental.pallas import tpu_sc as plsc

D = 128
E = 320000
N = 10000
R = 10000
DEG = 32

_NC = 2
_NS = 16
_NW = _NC * _NS

_CH = 80


def _ln(y, gamma, beta):
    mu = jnp.mean(y, axis=-1, keepdims=True)
    var = jnp.mean((y - mu) ** 2, axis=-1, keepdims=True)
    yn = (y - mu) / jnp.sqrt(var + 1e-5)
    return yn * gamma + beta


def _precompute_body(rect_ref, node_ref, w1t_ref, w2t_ref, at_ref, wggt_ref,
                     ggg_ref, bgg_ref, p_ref, q_ref, na_ref, rect_out_ref):
    rect = rect_ref[...]
    node = node_ref[...]
    p_ref[...] = jnp.dot(rect, w1t_ref[...], preferred_element_type=jnp.float32)
    q_ref[...] = jnp.dot(node, w2t_ref[...], preferred_element_type=jnp.float32)
    na_ref[...] = jnp.dot(node, at_ref[...], preferred_element_type=jnp.float32)
    y = jnp.tanh(jnp.dot(rect, wggt_ref[...], preferred_element_type=jnp.float32))
    rect_out_ref[...] = rect + _ln(y, ggg_ref[...], bgg_ref[...])


def _edge_body(bond_ref, s_ref, w0t_ref, g_ref, b_ref,
               db_ref, bout_ref):
    pre = jnp.dot(bond_ref[...], w0t_ref[...], preferred_element_type=jnp.float32)
    pre = pre + s_ref[...]
    db = _ln(jnp.tanh(pre), g_ref[...], b_ref[...])
    db_ref[...] = db
    bout_ref[...] = bond_ref[...] + db


def _node_body(g2_ref, coef_ref, node_ref, na_ref, bt_ref, g_ref, b_ref,
               nout_ref):
    acc = g2_ref[0] * coef_ref[0]
    for k in range(1, DEG):
        acc = acc + g2_ref[k] * coef_ref[k]
    agg = acc * (1.0 / DEG)
    pre = na_ref[...] + jnp.dot(agg, bt_ref[...], preferred_element_type=jnp.float32)
    dn = _ln(jnp.tanh(pre), g_ref[...], b_ref[...])
    nout_ref[...] = node_ref[...] + dn


_BN_A = 1000
_BE = 8000
_BN_C = 400


def _tc_precompute(rect, node, w1t, w2t, at, wggt, ggg, bgg):
    nblk = R // _BN_A
    row_spec = pl.BlockSpec((_BN_A, D), lambda i: (i, 0))
    full_spec = pl.BlockSpec((D, D), lambda i: (0, 0))
    vec_spec = pl.BlockSpec((1, D), lambda i: (0, 0))
    return pl.pallas_call(
        _precompute_body,
        grid=(nblk,),
        in_specs=[row_spec, row_spec, full_spec, full_spec, full_spec,
                  full_spec, vec_spec, vec_spec],
        out_specs=[row_spec, row_spec, row_spec, row_spec],
        out_shape=[jax.ShapeDtypeStruct((R, D), jnp.float32)] * 4,
    )(rect, node, w1t, w2t, at, wggt, ggg, bgg)


def _tc_edge(bond, s, w0t, g, b):
    nblk = E // _BE
    row_spec = pl.BlockSpec((_BE, D), lambda i: (i, 0))
    full_spec = pl.BlockSpec((D, D), lambda i: (0, 0))
    vec_spec = pl.BlockSpec((1, D), lambda i: (0, 0))
    return pl.pallas_call(
        _edge_body,
        grid=(nblk,),
        in_specs=[row_spec, row_spec, full_spec, vec_spec, vec_spec],
        out_specs=[row_spec, row_spec],
        out_shape=[jax.ShapeDtypeStruct((E, D), jnp.float32)] * 2,
    )(bond, s, w0t, g, b)


def _tc_node(g2, coef3, node, na, bt, g, b):
    nblk = N // _BN_C
    g2_spec = pl.BlockSpec((DEG, _BN_C, D), lambda i: (0, i, 0))
    coef_spec = pl.BlockSpec((DEG, _BN_C, 1), lambda i: (0, i, 0))
    row_spec = pl.BlockSpec((_BN_C, D), lambda i: (i, 0))
    full_spec = pl.BlockSpec((D, D), lambda i: (0, 0))
    vec_spec = pl.BlockSpec((1, D), lambda i: (0, 0))
    return pl.pallas_call(
        _node_body,
        grid=(nblk,),
        in_specs=[g2_spec, coef_spec, row_spec, row_spec, full_spec,
                  vec_spec, vec_spec],
        out_specs=row_spec,
        out_shape=jax.ShapeDtypeStruct((N, D), jnp.float32),
    )(g2, coef3, node, na, bt, g, b)


_NCHUNK = (E // _NW) // _CH


@functools.cache
def _sc_kernels():
    mesh = plsc.VectorSubcoreMesh(core_axis_name="c", subcore_axis_name="s",
                                  num_cores=_NC)

    @functools.partial(
        pl.kernel,
        mesh=mesh,
        out_type=jax.ShapeDtypeStruct((E, D), jnp.float32),
        scratch_types=[
            pltpu.VMEM((_NCHUNK, _CH), jnp.int32),
            pltpu.VMEM((_NCHUNK, _CH), jnp.int32),
            pltpu.VMEM((_CH, D), jnp.float32),
            pltpu.VMEM((_CH, D), jnp.float32),
            pltpu.VMEM((_CH, D), jnp.float32),
            pltpu.VMEM((_CH, D), jnp.float32),
            pltpu.SemaphoreType.DMA,
            pltpu.SemaphoreType.DMA,
            pltpu.SemaphoreType.DMA,
        ],
    )
    def _sc_gather_pq(p_hbm, q_hbm, src3, dst3, s_hbm,
                      idx_s, idx_d, bp0, bq0, bp1, bq1, gsp, gsq, ss):
        wid = lax.axis_index("s") * _NC + lax.axis_index("c")
        nper = E // _NW
        pltpu.sync_copy(src3.at[wid], idx_s)
        pltpu.sync_copy(dst3.at[wid], idx_d)

        def issue_gather(i, bp, bq):
            pltpu.async_copy(p_hbm.at[idx_s.at[i]], bp, gsp)
            pltpu.async_copy(q_hbm.at[idx_d.at[i]], bq, gsq)

        def wait_gather(bp, bq):
            pltpu.make_async_copy(p_hbm.at[pl.ds(0, _CH)], bp, gsp).wait()
            pltpu.make_async_copy(q_hbm.at[pl.ds(0, _CH)], bq, gsq).wait()

        def wait_scatter():
            pltpu.make_async_copy(bq0, s_hbm.at[pl.ds(0, _CH)], ss).wait()

        def step(i, bp, bq, bpn, bqn):
            @pl.when(i + 1 < _NCHUNK)
            def _():
                @pl.when(i >= 1)
                def _():
                    wait_scatter()
                issue_gather(i + 1, bpn, bqn)

            wait_gather(bp, bq)

            @plsc.parallel_loop(0, _CH, 1, unroll=2)
            def _(r):
                for j in range(D // 16):
                    sl = pl.ds(j * 16, 16)
                    bq[r, sl] = bq[r, sl] + bp[r, sl]

            pltpu.async_copy(bq, s_hbm.at[pl.ds(wid * nper + i * _CH, _CH)],
                             ss)

        issue_gather(0, bp0, bq0)

        def body(i2, carry):
            step(2 * i2, bp0, bq0, bp1, bq1)
            step(2 * i2 + 1, bp1, bq1, bp0, bq0)
            return carry

        lax.fori_loop(0, _NCHUNK // 2, body, 0)
        step(_NCHUNK - 1, bp0, bq0, bp1, bq1)
        wait_scatter()
        wait_scatter()

    @functools.partial(
        pl.kernel,
        mesh=mesh,
        out_type=jax.ShapeDtypeStruct((DEG * N, D), jnp.float32),
        scratch_types=[
            pltpu.VMEM((_NCHUNK, _CH), jnp.int32),
            pltpu.VMEM((2, _CH, D), jnp.float32),
            pltpu.SemaphoreType.DMA,
            pltpu.SemaphoreType.DMA,
        ],
    )
    def _sc_gather_deg(db_hbm, eidt3, g2_hbm, idx_v, buf, gs, ss):
        wid = lax.axis_index("s") * _NC + lax.axis_index("c")
        pltpu.sync_copy(eidt3.at[wid], idx_v)

        def body(i, carry):
            b = lax.rem(i, 2)
            nb = lax.rem(i + 1, 2)

            @pl.when(i + 1 < _NCHUNK)
            def _():
                @pl.when(i >= 1)
                def _():
                    pltpu.make_async_copy(buf.at[nb], g2_hbm.at[pl.ds(0, _CH)],
                                          ss).wait()
                pltpu.async_copy(db_hbm.at[idx_v.at[i + 1]], buf.at[nb], gs)

            pltpu.make_async_copy(db_hbm.at[pl.ds(0, _CH)], buf.at[b], gs).wait()
            base = wid * N + i * _CH
            pltpu.async_copy(buf.at[b], g2_hbm.at[pl.ds(base, _CH)], ss)
            return carry

        pltpu.async_copy(db_hbm.at[idx_v.at[0]], buf.at[0], gs)
        lax.fori_loop(0, _NCHUNK, body, 0)
        pltpu.make_async_copy(buf.at[0], g2_hbm.at[pl.ds(0, _CH)], ss).wait()
        pltpu.make_async_copy(buf.at[1], g2_hbm.at[pl.ds(0, _CH)], ss).wait()

    return _sc_gather_pq, _sc_gather_deg


def kernel(grid_mesh_bond_embedding, grid_rect_embedding, mesh_node_embedding,
           G2M_edge_id2pair_tensor, G2M_edge_id_of_node_tensor,
           G2M_edge_coef_node_tensor, W_GM2E, g_GM2E, b_GM2E,
           W_E2M, g_E2M, b_E2M, W_G2G, g_G2G, b_G2G):
    bond = grid_mesh_bond_embedding.reshape(E, D)
    rect = grid_rect_embedding.reshape(R, D)
    node = mesh_node_embedding.reshape(N, D)
    src3 = G2M_edge_id2pair_tensor[:, 0].reshape(_NW, _NCHUNK, _CH)
    dst3 = G2M_edge_id2pair_tensor[:, 1].reshape(_NW, _NCHUNK, _CH)
    eidt3 = G2M_edge_id_of_node_tensor.T.reshape(_NW, _NCHUNK, _CH)
    coef3 = G2M_edge_coef_node_tensor.transpose(1, 0, 2)

    w0t = W_GM2E[:, :D].T
    w1t = W_GM2E[:, D:2 * D].T
    w2t = W_GM2E[:, 2 * D:].T
    at = W_E2M[:, :D].T
    bt = W_E2M[:, D:].T
    wggt = W_G2G.T
    g_gm = g_GM2E.reshape(1, D); b_gm = b_GM2E.reshape(1, D)
    g_em = g_E2M.reshape(1, D); b_em = b_E2M.reshape(1, D)
    g_gg = g_G2G.reshape(1, D); b_gg = b_G2G.reshape(1, D)

    sc_gather_pq, sc_gather_deg = _sc_kernels()
    p, q, na, rect_out = _tc_precompute(rect, node, w1t, w2t, at, wggt,
                                        g_gg, b_gg)
    s = sc_gather_pq(p, q, src3, dst3)
    db, bond_out = _tc_edge(bond, s, w0t, g_gm, b_gm)
    g2 = sc_gather_deg(db, eidt3).reshape(DEG, N, D)
    node_out = _tc_node(g2, coef3, node, na, bt, g_em, b_em)

    return (bond_out.reshape(1, E, D),
            rect_out.reshape(1, R, D),
            node_out.reshape(1, N, D))
